# Initial kernel scaffold; baseline (speedup 1.0000x reference)
import jax
import jax.numpy as jnp
from jax.experimental import pallas as pl


def kernel(feat, edge_index, efeat, W1, b1, W2, b2):
    raise NotImplementedError("write your pallas kernel here")


# speedup vs baseline: 6.9218x; 6.9218x over previous
import functools

import jax
import jax.numpy as jnp
from jax import lax
from jax.experimental import pallas as pl
from jax.experimental.pallas import tpu as pltpu
from jax.experimental.pallas import tpu_sc as plsc

N = 10000
E = 320000
D = 128
D2 = D // 2

NC = 2
NS = 16
NW = NC * NS
B = 40
NROWS = E // B
RPT = NROWS // NS
RPW = NROWS // NW
NPAD = 10112
CH = NPAD // NS
ZC = 40
ZREM = CH - (CH // ZC) * ZC
T_TRI = (RPT - 2) // 3

_mesh = plsc.VectorSubcoreMesh(core_axis_name="c", subcore_axis_name="s")


@functools.partial(
    pl.kernel,
    out_type=(
        jax.ShapeDtypeStruct((NPAD,), jnp.float32),
        jax.ShapeDtypeStruct((NPAD,), jnp.float32),
        jax.ShapeDtypeStruct((NPAD,), jnp.float32),
        jax.ShapeDtypeStruct((NPAD,), jnp.float32),
    ),
    mesh=_mesh,
    scratch_types=[
        pltpu.VMEM((RPW, B), jnp.int32),
        pltpu.VMEM((640,), jnp.float32),
        pltpu.VMEM_SHARED((NPAD,), jnp.float32),
        pltpu.VMEM_SHARED((NPAD,), jnp.float32),
    ],
)
def _sc_degrees(src_hbm, dst_hbm, dego0_hbm, dego1_hbm, degi0_hbm, degi1_hbm,
                idx_v, buf_v, hout_s, hin_s):
    c = lax.axis_index("c")
    s = lax.axis_index("s")
    for k in range(40):
        buf_v[pl.ds(k * 16, 16)] = jnp.zeros((16,), jnp.float32)
    pltpu.sync_copy(buf_v.at[pl.ds(0, CH)], hout_s.at[pl.ds(s * CH, CH)])
    pltpu.sync_copy(buf_v.at[pl.ds(0, CH)], hin_s.at[pl.ds(s * CH, CH)])
    for k in range(4):
        buf_v[pl.ds(k * 16, 16)] = jnp.full((16,), 1.0, jnp.float32)
    plsc.subcore_barrier()

    w = c * NS + s
    pltpu.sync_copy(src_hbm.at[w], idx_v)

    def _body_src(j, carry):
        pltpu.sync_copy(buf_v.at[pl.ds(0, B)], hout_s.at[idx_v.at[j]],
                        add=True)
        return carry

    lax.fori_loop(0, RPW, _body_src, 0)

    pltpu.sync_copy(dst_hbm.at[w], idx_v)

    def _body_dst(j, carry):
        pltpu.sync_copy(buf_v.at[pl.ds(0, B)], hin_s.at[idx_v.at[j]],
                        add=True)
        return carry

    lax.fori_loop(0, RPW, _body_dst, 0)
    plsc.subcore_barrier()

    pltpu.sync_copy(hout_s.at[pl.ds(s * CH, CH)], buf_v.at[pl.ds(0, CH)])

    @pl.when(c == 0)
    def _():
        pltpu.sync_copy(buf_v.at[pl.ds(0, CH)], dego0_hbm.at[pl.ds(s * CH, CH)])

    @pl.when(c == 1)
    def _():
        pltpu.sync_copy(buf_v.at[pl.ds(0, CH)], dego1_hbm.at[pl.ds(s * CH, CH)])

    pltpu.sync_copy(hin_s.at[pl.ds(s * CH, CH)], buf_v.at[pl.ds(0, CH)])

    @pl.when(c == 0)
    def _():
        pltpu.sync_copy(buf_v.at[pl.ds(0, CH)], degi0_hbm.at[pl.ds(s * CH, CH)])

    @pl.when(c == 1)
    def _():
        pltpu.sync_copy(buf_v.at[pl.ds(0, CH)], degi1_hbm.at[pl.ds(s * CH, CH)])


@functools.partial(
    pl.kernel,
    out_type=jax.ShapeDtypeStruct((NC, NPAD, D2), jnp.float32),
    mesh=_mesh,
    scratch_types=[
        pltpu.VMEM((RPT * B,), jnp.int32),
        pltpu.VMEM((RPT, B), jnp.int32),
        pltpu.VMEM((B, D2), jnp.float32),
        pltpu.VMEM((B, D2), jnp.float32),
        pltpu.VMEM((B, D2), jnp.float32),
        pltpu.SemaphoreType.DMA,
        pltpu.SemaphoreType.DMA,
        pltpu.SemaphoreType.DMA,
        pltpu.VMEM_SHARED((NPAD, D2), jnp.float32),
    ],
    compiler_params=pltpu.CompilerParams(use_tc_tiling_on_sc=False),
)
def _sc_aggregate(h_hbm, srcf_hbm, dst_hbm, out_hbm, sidx, didx,
                  r_a, r_b, r_c, sem_a, sem_b, sem_c, agg_s):
    c = lax.axis_index("c")
    s = lax.axis_index("s")
    for r in range(B):
        for k in range(D2 // 16):
            r_a[r, pl.ds(k * 16, 16)] = jnp.zeros((16,), jnp.float32)
    for t in range(CH // ZC):
        pltpu.sync_copy(r_a.at[pl.ds(0, ZC)],
                        agg_s.at[pl.ds(s * CH + t * ZC, ZC)])
    pltpu.sync_copy(r_a.at[pl.ds(0, ZREM)],
                    agg_s.at[pl.ds(s * CH + (CH // ZC) * ZC, ZREM)])
    plsc.subcore_barrier()

    pltpu.sync_copy(srcf_hbm.at[s], sidx)
    pltpu.sync_copy(dst_hbm.at[s], didx)

    cbias = jnp.full((16,), c * N, jnp.int32)

    def _adj(i, carry):
        sidx[pl.ds(i * 16, 16)] = sidx[pl.ds(i * 16, 16)] + cbias
        return carry

    lax.fori_loop(0, RPT * B // 16, _adj, 0)

    def _gather(j, buf, sem):
        return pltpu.async_copy(h_hbm.at[sidx.at[pl.ds(j * B, B)]], buf, sem)

    def _gwait(j, buf, sem):
        pltpu.make_async_copy(h_hbm.at[sidx.at[pl.ds(j * B, B)]], buf,
                              sem).wait()

    def _scat(j, buf):
        pltpu.sync_copy(buf, agg_s.at[didx.at[j]], add=True)

    _gather(0, r_a, sem_a)
    _gather(1, r_b, sem_b)

    def _tri(t, carry):
        j0 = 3 * t
        _gwait(j0, r_a, sem_a)
        _gather(j0 + 2, r_c, sem_c)
        _scat(j0, r_a)
        _gwait(j0 + 1, r_b, sem_b)
        _gather(j0 + 3, r_a, sem_a)
        _scat(j0 + 1, r_b)
        _gwait(j0 + 2, r_c, sem_c)
        _gather(j0 + 4, r_b, sem_b)
        _scat(j0 + 2, r_c)
        return carry

    lax.fori_loop(0, T_TRI, _tri, 0)
    _gwait(RPT - 2, r_a, sem_a)
    _scat(RPT - 2, r_a)
    _gwait(RPT - 1, r_b, sem_b)
    _scat(RPT - 1, r_b)

    acc = lax.fori_loop(0, 4096, lambda i, a: a + i, jnp.int32(0))
    sidx[pl.ds(0, 16)] = jnp.full((16,), acc, jnp.int32)
    plsc.subcore_barrier()

    for t in range(CH // ZC):
        pltpu.sync_copy(agg_s.at[pl.ds(s * CH + t * ZC, ZC)],
                        r_a.at[pl.ds(0, ZC)])
        pltpu.sync_copy(r_a.at[pl.ds(0, ZC)],
                        out_hbm.at[c, pl.ds(s * CH + t * ZC, ZC)])
    pltpu.sync_copy(agg_s.at[pl.ds(s * CH + (CH // ZC) * ZC, ZREM)],
                    r_a.at[pl.ds(0, ZREM)])
    pltpu.sync_copy(r_a.at[pl.ds(0, ZREM)],
                    out_hbm.at[c, pl.ds(s * CH + (CH // ZC) * ZC, ZREM)])


_BR = 400


def _norm(two_col):
    deg = two_col[:, 0:1] + two_col[:, 1:2]
    return jnp.where(deg > 0, lax.rsqrt(deg), 0.0)


def _split_store(o_ref, y):
    o_ref[0, :, :] = y[:, :D2]
    o_ref[1, :, :] = y[:, D2:]


def _mm1_body(f_ref, w_ref, dego_ref, o_ref):
    ns = _norm(dego_ref[...])
    y = jnp.dot(f_ref[...], w_ref[...],
                preferred_element_type=jnp.float32) * ns
    _split_store(o_ref, y)


def _tc_mm1(feat, W1, dego_t):
    return pl.pallas_call(
        _mm1_body,
        grid=(N // _BR,),
        in_specs=[
            pl.BlockSpec((_BR, D), lambda i: (i, 0)),
            pl.BlockSpec((D, D), lambda i: (0, 0)),
            pl.BlockSpec((_BR, NC), lambda i: (i, 0)),
        ],
        out_specs=pl.BlockSpec((NC, _BR, D2), lambda i: (0, i, 0)),
        out_shape=jax.ShapeDtypeStruct((NC, N, D2), jnp.float32),
    )(feat, W1, dego_t)


def _mm2_body(p_ref, degi_ref, dego_ref, b_ref, w_ref, o_ref):
    agg = jnp.concatenate([p_ref[0], p_ref[1]], axis=-1)
    nd = _norm(degi_ref[...])
    x = jnp.maximum(agg * nd + b_ref[...], 0.0)
    ns = _norm(dego_ref[...])
    y = jnp.dot(x, w_ref[...], preferred_element_type=jnp.float32) * ns
    _split_store(o_ref, y)


def _tc_mm2(P, degi_t, dego_t, b1, W2):
    return pl.pallas_call(
        _mm2_body,
        grid=(N // _BR,),
        in_specs=[
            pl.BlockSpec((NC, _BR, D2), lambda i: (0, i, 0)),
            pl.BlockSpec((_BR, NC), lambda i: (i, 0)),
            pl.BlockSpec((_BR, NC), lambda i: (i, 0)),
            pl.BlockSpec((1, D), lambda i: (0, 0)),
            pl.BlockSpec((D, D), lambda i: (0, 0)),
        ],
        out_specs=pl.BlockSpec((NC, _BR, D2), lambda i: (0, i, 0)),
        out_shape=jax.ShapeDtypeStruct((NC, N, D2), jnp.float32),
    )(P, degi_t, dego_t, b1, W2)


def _ep_body(p_ref, degi_ref, b_ref, o_ref):
    agg = jnp.concatenate([p_ref[0], p_ref[1]], axis=-1)
    nd = _norm(degi_ref[...])
    o_ref[...] = jnp.maximum(agg * nd + b_ref[...], 0.0)


def _tc_ep(P, degi_t, b2):
    return pl.pallas_call(
        _ep_body,
        grid=(N // _BR,),
        in_specs=[
            pl.BlockSpec((NC, _BR, D2), lambda i: (0, i, 0)),
            pl.BlockSpec((_BR, NC), lambda i: (i, 0)),
            pl.BlockSpec((1, D), lambda i: (0, 0)),
        ],
        out_specs=pl.BlockSpec((_BR, D), lambda i: (i, 0)),
        out_shape=jax.ShapeDtypeStruct((N, D), jnp.float32),
    )(P, degi_t, b2)


def kernel(feat, edge_index, efeat, W1, b1, W2, b2):
    del efeat
    srcf = edge_index[0].reshape(NS, RPT * B)
    dst3d = edge_index[1].reshape(NS, RPT, B)
    src_deg = edge_index[0].reshape(NW, RPW, B)
    dst_deg = edge_index[1].reshape(NW, RPW, B)

    dego0, dego1, degi0, degi1 = _sc_degrees(src_deg, dst_deg)
    dego_t = jnp.stack([dego0, dego1], axis=1)
    degi_t = jnp.stack([degi0, degi1], axis=1)

    h1 = _tc_mm1(feat, W1, dego_t)
    P1 = _sc_aggregate(h1.reshape(NC * N, D2), srcf, dst3d)
    h2 = _tc_mm2(P1, degi_t, dego_t, b1.reshape(1, D), W2)
    P2 = _sc_aggregate(h2.reshape(NC * N, D2), srcf, dst3d)
    return _tc_ep(P2, degi_t, b2.reshape(1, D))


# speedup vs baseline: 9.4728x; 1.3685x over previous
import functools

import jax
import jax.numpy as jnp
from jax import lax
from jax.experimental import pallas as pl
from jax.experimental.pallas import tpu as pltpu
from jax.experimental.pallas import tpu_sc as plsc

N = 10000
E = 320000
D = 128
D2 = D // 2

NC = 2
NS = 16
NW = NC * NS
B = 80
NROWS = E // B
RPT = NROWS // NS
RPW = NROWS // NW
NPAD = 10112
CH = NPAD // NS
ZC = 40
ZREM = CH - (CH // ZC) * ZC
T_TRI = (RPT - 2) // 3

_mesh = plsc.VectorSubcoreMesh(core_axis_name="c", subcore_axis_name="s")


@functools.partial(
    pl.kernel,
    out_type=(
        jax.ShapeDtypeStruct((NPAD,), jnp.float32),
        jax.ShapeDtypeStruct((NPAD,), jnp.float32),
        jax.ShapeDtypeStruct((NPAD,), jnp.float32),
        jax.ShapeDtypeStruct((NPAD,), jnp.float32),
    ),
    mesh=_mesh,
    scratch_types=[
        pltpu.VMEM((RPW, B), jnp.int32),
        pltpu.VMEM((640,), jnp.float32),
        pltpu.VMEM_SHARED((NPAD,), jnp.float32),
        pltpu.VMEM_SHARED((NPAD,), jnp.float32),
    ],
)
def _sc_degrees(src_hbm, dst_hbm, dego0_hbm, dego1_hbm, degi0_hbm, degi1_hbm,
                idx_v, buf_v, hout_s, hin_s):
    c = lax.axis_index("c")
    s = lax.axis_index("s")
    for k in range(40):
        buf_v[pl.ds(k * 16, 16)] = jnp.zeros((16,), jnp.float32)
    pltpu.sync_copy(buf_v.at[pl.ds(0, CH)], hout_s.at[pl.ds(s * CH, CH)])
    pltpu.sync_copy(buf_v.at[pl.ds(0, CH)], hin_s.at[pl.ds(s * CH, CH)])
    for k in range((B + 15) // 16):
        buf_v[pl.ds(k * 16, 16)] = jnp.full((16,), 1.0, jnp.float32)
    plsc.subcore_barrier()

    w = c * NS + s
    pltpu.sync_copy(src_hbm.at[w], idx_v)

    def _body_src(j, carry):
        pltpu.sync_copy(buf_v.at[pl.ds(0, B)], hout_s.at[idx_v.at[j]],
                        add=True)
        return carry

    lax.fori_loop(0, RPW, _body_src, 0)

    pltpu.sync_copy(dst_hbm.at[w], idx_v)

    def _body_dst(j, carry):
        pltpu.sync_copy(buf_v.at[pl.ds(0, B)], hin_s.at[idx_v.at[j]],
                        add=True)
        return carry

    lax.fori_loop(0, RPW, _body_dst, 0)
    plsc.subcore_barrier()

    pltpu.sync_copy(hout_s.at[pl.ds(s * CH, CH)], buf_v.at[pl.ds(0, CH)])

    @pl.when(c == 0)
    def _():
        pltpu.sync_copy(buf_v.at[pl.ds(0, CH)], dego0_hbm.at[pl.ds(s * CH, CH)])

    @pl.when(c == 1)
    def _():
        pltpu.sync_copy(buf_v.at[pl.ds(0, CH)], dego1_hbm.at[pl.ds(s * CH, CH)])

    pltpu.sync_copy(hin_s.at[pl.ds(s * CH, CH)], buf_v.at[pl.ds(0, CH)])

    @pl.when(c == 0)
    def _():
        pltpu.sync_copy(buf_v.at[pl.ds(0, CH)], degi0_hbm.at[pl.ds(s * CH, CH)])

    @pl.when(c == 1)
    def _():
        pltpu.sync_copy(buf_v.at[pl.ds(0, CH)], degi1_hbm.at[pl.ds(s * CH, CH)])


@functools.partial(
    pl.kernel,
    out_type=jax.ShapeDtypeStruct((NC, NPAD, D2), jnp.float32),
    mesh=_mesh,
    scratch_types=[
        pltpu.VMEM((RPT * B,), jnp.int32),
        pltpu.VMEM((RPT, B), jnp.int32),
        pltpu.VMEM((B, D2), jnp.float32),
        pltpu.VMEM((B, D2), jnp.float32),
        pltpu.VMEM((B, D2), jnp.float32),
        pltpu.SemaphoreType.DMA,
        pltpu.SemaphoreType.DMA,
        pltpu.SemaphoreType.DMA,
        pltpu.VMEM_SHARED((NPAD, D2), jnp.float32),
    ],
    compiler_params=pltpu.CompilerParams(use_tc_tiling_on_sc=False),
)
def _sc_aggregate(h_hbm, srcf_hbm, dst_hbm, out_hbm, sidx, didx,
                  r_a, r_b, r_c, sem_a, sem_b, sem_c, agg_s):
    c = lax.axis_index("c")
    s = lax.axis_index("s")
    for r in range(B):
        for k in range(D2 // 16):
            r_a[r, pl.ds(k * 16, 16)] = jnp.zeros((16,), jnp.float32)
    for t in range(CH // ZC):
        pltpu.sync_copy(r_a.at[pl.ds(0, ZC)],
                        agg_s.at[pl.ds(s * CH + t * ZC, ZC)])
    pltpu.sync_copy(r_a.at[pl.ds(0, ZREM)],
                    agg_s.at[pl.ds(s * CH + (CH // ZC) * ZC, ZREM)])
    plsc.subcore_barrier()

    pltpu.sync_copy(srcf_hbm.at[s], sidx)
    pltpu.sync_copy(dst_hbm.at[s], didx)

    cbias = jnp.full((16,), c * N, jnp.int32)

    def _adj(i, carry):
        sidx[pl.ds(i * 16, 16)] = sidx[pl.ds(i * 16, 16)] + cbias
        return carry

    lax.fori_loop(0, RPT * B // 16, _adj, 0)

    def _gather(j, buf, sem):
        return pltpu.async_copy(h_hbm.at[sidx.at[pl.ds(j * B, B)]], buf, sem)

    def _gwait(j, buf, sem):
        pltpu.make_async_copy(h_hbm.at[sidx.at[pl.ds(j * B, B)]], buf,
                              sem).wait()

    def _scat(j, buf):
        pltpu.sync_copy(buf, agg_s.at[didx.at[j]], add=True)

    _gather(0, r_a, sem_a)
    _gather(1, r_b, sem_b)

    def _tri(t, carry):
        j0 = 3 * t
        _gwait(j0, r_a, sem_a)
        _gather(j0 + 2, r_c, sem_c)
        _scat(j0, r_a)
        _gwait(j0 + 1, r_b, sem_b)
        _gather(j0 + 3, r_a, sem_a)
        _scat(j0 + 1, r_b)
        _gwait(j0 + 2, r_c, sem_c)
        _gather(j0 + 4, r_b, sem_b)
        _scat(j0 + 2, r_c)
        return carry

    lax.fori_loop(0, T_TRI, _tri, 0)
    _bufs = ((r_a, sem_a), (r_b, sem_b), (r_c, sem_c))
    for _k in range(RPT - 3 * T_TRI):
        _j = 3 * T_TRI + _k
        _gwait(_j, *_bufs[_j % 3])
        if _j + 2 >= 3 * T_TRI + 2 and _j + 2 < RPT:
            _gather(_j + 2, *_bufs[(_j + 2) % 3])
        _scat(_j, _bufs[_j % 3][0])

    acc = lax.fori_loop(0, 4096, lambda i, a: a + i, jnp.int32(0))
    sidx[pl.ds(0, 16)] = jnp.full((16,), acc, jnp.int32)
    plsc.subcore_barrier()

    for t in range(CH // ZC):
        pltpu.sync_copy(agg_s.at[pl.ds(s * CH + t * ZC, ZC)],
                        r_a.at[pl.ds(0, ZC)])
        pltpu.sync_copy(r_a.at[pl.ds(0, ZC)],
                        out_hbm.at[c, pl.ds(s * CH + t * ZC, ZC)])
    pltpu.sync_copy(agg_s.at[pl.ds(s * CH + (CH // ZC) * ZC, ZREM)],
                    r_a.at[pl.ds(0, ZREM)])
    pltpu.sync_copy(r_a.at[pl.ds(0, ZREM)],
                    out_hbm.at[c, pl.ds(s * CH + (CH // ZC) * ZC, ZREM)])


_BR = 400


def _norm(two_col):
    deg = two_col[:, 0:1] + two_col[:, 1:2]
    return jnp.where(deg > 0, lax.rsqrt(deg), 0.0)


def _split_store(o_ref, y):
    o_ref[0, :, :] = y[:, :D2]
    o_ref[1, :, :] = y[:, D2:]


def _mm1_body(f_ref, w_ref, dego_ref, o_ref):
    ns = _norm(dego_ref[...])
    y = jnp.dot(f_ref[...], w_ref[...],
                preferred_element_type=jnp.float32) * ns
    _split_store(o_ref, y)


def _tc_mm1(feat, W1, dego_t):
    return pl.pallas_call(
        _mm1_body,
        grid=(N // _BR,),
        in_specs=[
            pl.BlockSpec((_BR, D), lambda i: (i, 0)),
            pl.BlockSpec((D, D), lambda i: (0, 0)),
            pl.BlockSpec((_BR, NC), lambda i: (i, 0)),
        ],
        out_specs=pl.BlockSpec((NC, _BR, D2), lambda i: (0, i, 0)),
        out_shape=jax.ShapeDtypeStruct((NC, N, D2), jnp.float32),
    )(feat, W1, dego_t)


def _mm2_body(p_ref, degi_ref, dego_ref, b_ref, w_ref, o_ref):
    agg = jnp.concatenate([p_ref[0], p_ref[1]], axis=-1)
    nd = _norm(degi_ref[...])
    x = jnp.maximum(agg * nd + b_ref[...], 0.0)
    ns = _norm(dego_ref[...])
    y = jnp.dot(x, w_ref[...], preferred_element_type=jnp.float32) * ns
    _split_store(o_ref, y)


def _tc_mm2(P, degi_t, dego_t, b1, W2):
    return pl.pallas_call(
        _mm2_body,
        grid=(N // _BR,),
        in_specs=[
            pl.BlockSpec((NC, _BR, D2), lambda i: (0, i, 0)),
            pl.BlockSpec((_BR, NC), lambda i: (i, 0)),
            pl.BlockSpec((_BR, NC), lambda i: (i, 0)),
            pl.BlockSpec((1, D), lambda i: (0, 0)),
            pl.BlockSpec((D, D), lambda i: (0, 0)),
        ],
        out_specs=pl.BlockSpec((NC, _BR, D2), lambda i: (0, i, 0)),
        out_shape=jax.ShapeDtypeStruct((NC, N, D2), jnp.float32),
    )(P, degi_t, dego_t, b1, W2)


def _ep_body(p_ref, degi_ref, b_ref, o_ref):
    agg = jnp.concatenate([p_ref[0], p_ref[1]], axis=-1)
    nd = _norm(degi_ref[...])
    o_ref[...] = jnp.maximum(agg * nd + b_ref[...], 0.0)


def _tc_ep(P, degi_t, b2):
    return pl.pallas_call(
        _ep_body,
        grid=(N // _BR,),
        in_specs=[
            pl.BlockSpec((NC, _BR, D2), lambda i: (0, i, 0)),
            pl.BlockSpec((_BR, NC), lambda i: (i, 0)),
            pl.BlockSpec((1, D), lambda i: (0, 0)),
        ],
        out_specs=pl.BlockSpec((_BR, D), lambda i: (i, 0)),
        out_shape=jax.ShapeDtypeStruct((N, D), jnp.float32),
    )(P, degi_t, b2)


def kernel(feat, edge_index, efeat, W1, b1, W2, b2):
    del efeat
    srcf = edge_index[0].reshape(NS, RPT * B)
    dst3d = edge_index[1].reshape(NS, RPT, B)
    src_deg = edge_index[0].reshape(NW, RPW, B)
    dst_deg = edge_index[1].reshape(NW, RPW, B)

    dego0, dego1, degi0, degi1 = _sc_degrees(src_deg, dst_deg)
    dego_t = jnp.stack([dego0, dego1], axis=1)
    degi_t = jnp.stack([degi0, degi1], axis=1)

    h1 = _tc_mm1(feat, W1, dego_t)
    P1 = _sc_aggregate(h1.reshape(NC * N, D2), srcf, dst3d)
    h2 = _tc_mm2(P1, degi_t, dego_t, b1.reshape(1, D), W2)
    P2 = _sc_aggregate(h2.reshape(NC * N, D2), srcf, dst3d)
    return _tc_ep(P2, degi_t, b2.reshape(1, D))


# speedup vs baseline: 9.9539x; 1.0508x over previous
import functools

import jax
import jax.numpy as jnp
from jax import lax
from jax.experimental import pallas as pl
from jax.experimental.pallas import tpu as pltpu
from jax.experimental.pallas import tpu_sc as plsc

N = 10000
E = 320000
D = 128
D2 = D // 2

NC = 2
NS = 16
NW = NC * NS
B = 80
NROWS = E // B
RPT = NROWS // NS
RPW = NROWS // NW
NPAD = 10112
CH = NPAD // NS
ZC = 40
ZREM = CH - (CH // ZC) * ZC
T_TRI = (RPT - 2) // 3

_mesh = plsc.VectorSubcoreMesh(core_axis_name="c", subcore_axis_name="s")


@functools.partial(
    pl.kernel,
    out_type=(
        jax.ShapeDtypeStruct((NPAD,), jnp.float32),
        jax.ShapeDtypeStruct((NPAD,), jnp.float32),
        jax.ShapeDtypeStruct((NPAD,), jnp.float32),
        jax.ShapeDtypeStruct((NPAD,), jnp.float32),
    ),
    mesh=_mesh,
    scratch_types=[
        pltpu.VMEM((RPW, B), jnp.int32),
        pltpu.VMEM((640,), jnp.float32),
        pltpu.VMEM_SHARED((NPAD,), jnp.float32),
        pltpu.VMEM_SHARED((NPAD,), jnp.float32),
    ],
)
def _sc_degrees(src_hbm, dst_hbm, dego0_hbm, dego1_hbm, degi0_hbm, degi1_hbm,
                idx_v, buf_v, hout_s, hin_s):
    c = lax.axis_index("c")
    s = lax.axis_index("s")
    for k in range(40):
        buf_v[pl.ds(k * 16, 16)] = jnp.zeros((16,), jnp.float32)
    pltpu.sync_copy(buf_v.at[pl.ds(0, CH)], hout_s.at[pl.ds(s * CH, CH)])
    pltpu.sync_copy(buf_v.at[pl.ds(0, CH)], hin_s.at[pl.ds(s * CH, CH)])
    for k in range((B + 15) // 16):
        buf_v[pl.ds(k * 16, 16)] = jnp.full((16,), 1.0, jnp.float32)
    plsc.subcore_barrier()

    w = c * NS + s
    pltpu.sync_copy(src_hbm.at[w], idx_v)

    def _body_src(j, carry):
        pltpu.sync_copy(buf_v.at[pl.ds(0, B)], hout_s.at[idx_v.at[j]],
                        add=True)
        return carry

    lax.fori_loop(0, RPW, _body_src, 0)

    pltpu.sync_copy(dst_hbm.at[w], idx_v)

    def _body_dst(j, carry):
        pltpu.sync_copy(buf_v.at[pl.ds(0, B)], hin_s.at[idx_v.at[j]],
                        add=True)
        return carry

    lax.fori_loop(0, RPW, _body_dst, 0)
    plsc.subcore_barrier()

    pltpu.sync_copy(hout_s.at[pl.ds(s * CH, CH)], buf_v.at[pl.ds(0, CH)])

    @pl.when(c == 0)
    def _():
        pltpu.sync_copy(buf_v.at[pl.ds(0, CH)], dego0_hbm.at[pl.ds(s * CH, CH)])

    @pl.when(c == 1)
    def _():
        pltpu.sync_copy(buf_v.at[pl.ds(0, CH)], dego1_hbm.at[pl.ds(s * CH, CH)])

    pltpu.sync_copy(hin_s.at[pl.ds(s * CH, CH)], buf_v.at[pl.ds(0, CH)])

    @pl.when(c == 0)
    def _():
        pltpu.sync_copy(buf_v.at[pl.ds(0, CH)], degi0_hbm.at[pl.ds(s * CH, CH)])

    @pl.when(c == 1)
    def _():
        pltpu.sync_copy(buf_v.at[pl.ds(0, CH)], degi1_hbm.at[pl.ds(s * CH, CH)])


@functools.partial(
    pl.kernel,
    out_type=jax.ShapeDtypeStruct((NC, NPAD, D2), jnp.float32),
    mesh=_mesh,
    scratch_types=[
        pltpu.VMEM((RPT * B,), jnp.int32),
        pltpu.VMEM((RPT, B), jnp.int32),
        pltpu.VMEM((B, D2), jnp.float32),
        pltpu.VMEM((B, D2), jnp.float32),
        pltpu.VMEM((B, D2), jnp.float32),
        pltpu.SemaphoreType.DMA,
        pltpu.SemaphoreType.DMA,
        pltpu.SemaphoreType.DMA,
        (pltpu.SemaphoreType.DMA, pltpu.SemaphoreType.DMA,
         pltpu.SemaphoreType.DMA),
        pltpu.VMEM_SHARED((NPAD, D2), jnp.float32),
    ],
    compiler_params=pltpu.CompilerParams(use_tc_tiling_on_sc=False),
)
def _sc_aggregate(h_hbm, srcf_hbm, dst_hbm, out_hbm, sidx, didx,
                  r_a, r_b, r_c, sem_a, sem_b, sem_c, ssems, agg_s):
    c = lax.axis_index("c")
    s = lax.axis_index("s")
    for r in range(B):
        for k in range(D2 // 16):
            r_a[r, pl.ds(k * 16, 16)] = jnp.zeros((16,), jnp.float32)
    for t in range(CH // ZC):
        pltpu.sync_copy(r_a.at[pl.ds(0, ZC)],
                        agg_s.at[pl.ds(s * CH + t * ZC, ZC)])
    pltpu.sync_copy(r_a.at[pl.ds(0, ZREM)],
                    agg_s.at[pl.ds(s * CH + (CH // ZC) * ZC, ZREM)])
    plsc.subcore_barrier()

    pltpu.sync_copy(srcf_hbm.at[s], sidx)
    pltpu.sync_copy(dst_hbm.at[s], didx)

    cbias = jnp.full((16,), c * N, jnp.int32)

    def _adj(i, carry):
        sidx[pl.ds(i * 16, 16)] = sidx[pl.ds(i * 16, 16)] + cbias
        return carry

    lax.fori_loop(0, RPT * B // 16, _adj, 0)

    _bufs = (r_a, r_b, r_c)
    _gsems = (sem_a, sem_b, sem_c)

    def _gather(j, x):
        pltpu.async_copy(h_hbm.at[sidx.at[pl.ds(j * B, B)]], _bufs[x],
                         _gsems[x])

    def _gwait(j, x):
        pltpu.make_async_copy(h_hbm.at[sidx.at[pl.ds(j * B, B)]], _bufs[x],
                              _gsems[x]).wait()

    def _sstart(j, x):
        pltpu.make_async_copy(_bufs[x], agg_s.at[didx.at[j]],
                              ssems[x]).start(add=True)

    def _swait(j, x):
        pltpu.make_async_copy(_bufs[x], agg_s.at[didx.at[j]],
                              ssems[x]).wait()

    _gather(0, 0)
    _gwait(0, 0)
    _sstart(0, 0)
    _gather(1, 1)
    _gwait(1, 1)
    _sstart(1, 1)
    _gather(2, 2)

    def _step(j, x):
        _swait(j - 2, (x + 1) % 3)
        _gather(j + 1, (x + 1) % 3)
        _gwait(j, x)
        _sstart(j, x)

    def _tri(t, carry):
        j0 = 3 * t + 2
        _step(j0, 2)
        _step(j0 + 1, 0)
        _step(j0 + 2, 1)
        return carry

    lax.fori_loop(0, (RPT - 4) // 3, _tri, 0)
    for _j in range(3 * ((RPT - 4) // 3) + 2, RPT):
        _x = _j % 3
        _swait(_j - 2, (_x + 1) % 3)
        if _j + 1 < RPT:
            _gather(_j + 1, (_x + 1) % 3)
        _gwait(_j, _x)
        _sstart(_j, _x)
    _swait(RPT - 2, (RPT - 2) % 3)
    _swait(RPT - 1, (RPT - 1) % 3)

    acc = lax.fori_loop(0, 2048, lambda i, a: a + i, jnp.int32(0))
    sidx[pl.ds(0, 16)] = jnp.full((16,), acc, jnp.int32)
    plsc.subcore_barrier()

    for t in range(CH // ZC):
        pltpu.sync_copy(agg_s.at[pl.ds(s * CH + t * ZC, ZC)],
                        r_a.at[pl.ds(0, ZC)])
        pltpu.sync_copy(r_a.at[pl.ds(0, ZC)],
                        out_hbm.at[c, pl.ds(s * CH + t * ZC, ZC)])
    pltpu.sync_copy(agg_s.at[pl.ds(s * CH + (CH // ZC) * ZC, ZREM)],
                    r_a.at[pl.ds(0, ZREM)])
    pltpu.sync_copy(r_a.at[pl.ds(0, ZREM)],
                    out_hbm.at[c, pl.ds(s * CH + (CH // ZC) * ZC, ZREM)])


_BR = 1000


def _norm(two_col):
    deg = two_col[:, 0:1] + two_col[:, 1:2]
    return jnp.where(deg > 0, lax.rsqrt(deg), 0.0)


def _split_store(o_ref, y):
    o_ref[0, :, :] = y[:, :D2]
    o_ref[1, :, :] = y[:, D2:]


def _mm1_body(f_ref, w_ref, dego_ref, o_ref):
    ns = _norm(dego_ref[...])
    y = jnp.dot(f_ref[...], w_ref[...],
                preferred_element_type=jnp.float32) * ns
    _split_store(o_ref, y)


def _tc_mm1(feat, W1, dego_t):
    return pl.pallas_call(
        _mm1_body,
        grid=(N // _BR,),
        in_specs=[
            pl.BlockSpec((_BR, D), lambda i: (i, 0)),
            pl.BlockSpec((D, D), lambda i: (0, 0)),
            pl.BlockSpec((_BR, NC), lambda i: (i, 0)),
        ],
        out_specs=pl.BlockSpec((NC, _BR, D2), lambda i: (0, i, 0)),
        out_shape=jax.ShapeDtypeStruct((NC, N, D2), jnp.float32),
    )(feat, W1, dego_t)


def _mm2_body(p_ref, degi_ref, dego_ref, b_ref, w_ref, o_ref):
    agg = jnp.concatenate([p_ref[0], p_ref[1]], axis=-1)
    nd = _norm(degi_ref[...])
    x = jnp.maximum(agg * nd + b_ref[...], 0.0)
    ns = _norm(dego_ref[...])
    y = jnp.dot(x, w_ref[...], preferred_element_type=jnp.float32) * ns
    _split_store(o_ref, y)


def _tc_mm2(P, degi_t, dego_t, b1, W2):
    return pl.pallas_call(
        _mm2_body,
        grid=(N // _BR,),
        in_specs=[
            pl.BlockSpec((NC, _BR, D2), lambda i: (0, i, 0)),
            pl.BlockSpec((_BR, NC), lambda i: (i, 0)),
            pl.BlockSpec((_BR, NC), lambda i: (i, 0)),
            pl.BlockSpec((1, D), lambda i: (0, 0)),
            pl.BlockSpec((D, D), lambda i: (0, 0)),
        ],
        out_specs=pl.BlockSpec((NC, _BR, D2), lambda i: (0, i, 0)),
        out_shape=jax.ShapeDtypeStruct((NC, N, D2), jnp.float32),
    )(P, degi_t, dego_t, b1, W2)


def _ep_body(p_ref, degi_ref, b_ref, o_ref):
    agg = jnp.concatenate([p_ref[0], p_ref[1]], axis=-1)
    nd = _norm(degi_ref[...])
    o_ref[...] = jnp.maximum(agg * nd + b_ref[...], 0.0)


def _tc_ep(P, degi_t, b2):
    return pl.pallas_call(
        _ep_body,
        grid=(N // _BR,),
        in_specs=[
            pl.BlockSpec((NC, _BR, D2), lambda i: (0, i, 0)),
            pl.BlockSpec((_BR, NC), lambda i: (i, 0)),
            pl.BlockSpec((1, D), lambda i: (0, 0)),
        ],
        out_specs=pl.BlockSpec((_BR, D), lambda i: (i, 0)),
        out_shape=jax.ShapeDtypeStruct((N, D), jnp.float32),
    )(P, degi_t, b2)


def kernel(feat, edge_index, efeat, W1, b1, W2, b2):
    del efeat
    srcf = edge_index[0].reshape(NS, RPT * B)
    dst3d = edge_index[1].reshape(NS, RPT, B)
    src_deg = edge_index[0].reshape(NW, RPW, B)
    dst_deg = edge_index[1].reshape(NW, RPW, B)

    dego0, dego1, degi0, degi1 = _sc_degrees(src_deg, dst_deg)
    dego_t = jnp.stack([dego0, dego1], axis=1)
    degi_t = jnp.stack([degi0, degi1], axis=1)

    h1 = _tc_mm1(feat, W1, dego_t)
    P1 = _sc_aggregate(h1.reshape(NC * N, D2), srcf, dst3d)
    h2 = _tc_mm2(P1, degi_t, dego_t, b1.reshape(1, D), W2)
    P2 = _sc_aggregate(h2.reshape(NC * N, D2), srcf, dst3d)
    return _tc_ep(P2, degi_t, b2.reshape(1, D))


# speedup vs baseline: 10.3408x; 1.0389x over previous
import functools

import jax
import jax.numpy as jnp
from jax import lax
from jax.experimental import pallas as pl
from jax.experimental.pallas import tpu as pltpu
from jax.experimental.pallas import tpu_sc as plsc

N = 10000
E = 320000
D = 128
D2 = D // 2

NC = 2
NS = 16
NW = NC * NS
B = 80
NROWS = E // B
RPT = NROWS // NS
RPW = NROWS // NW
NPAD = 10112
CH = NPAD // NS
ZC = 40
ZREM = CH - (CH // ZC) * ZC
T_TRI = (RPT - 2) // 3

_mesh = plsc.VectorSubcoreMesh(core_axis_name="c", subcore_axis_name="s")


@functools.partial(
    pl.kernel,
    out_type=(
        jax.ShapeDtypeStruct((NPAD,), jnp.float32),
        jax.ShapeDtypeStruct((NPAD,), jnp.float32),
        jax.ShapeDtypeStruct((NPAD,), jnp.float32),
        jax.ShapeDtypeStruct((NPAD,), jnp.float32),
    ),
    mesh=_mesh,
    scratch_types=[
        pltpu.VMEM((RPW, B), jnp.int32),
        pltpu.VMEM((640,), jnp.float32),
        (pltpu.SemaphoreType.DMA, pltpu.SemaphoreType.DMA,
         pltpu.SemaphoreType.DMA, pltpu.SemaphoreType.DMA),
        pltpu.VMEM_SHARED((NPAD,), jnp.float32),
        pltpu.VMEM_SHARED((NPAD,), jnp.float32),
    ],
)
def _sc_degrees(src_hbm, dst_hbm, dego0_hbm, dego1_hbm, degi0_hbm, degi1_hbm,
                idx_v, buf_v, dsems, hout_s, hin_s):
    c = lax.axis_index("c")
    s = lax.axis_index("s")
    for k in range(40):
        buf_v[pl.ds(k * 16, 16)] = jnp.zeros((16,), jnp.float32)
    pltpu.sync_copy(buf_v.at[pl.ds(0, CH)], hout_s.at[pl.ds(s * CH, CH)])
    pltpu.sync_copy(buf_v.at[pl.ds(0, CH)], hin_s.at[pl.ds(s * CH, CH)])
    for k in range((B + 15) // 16):
        buf_v[pl.ds(k * 16, 16)] = jnp.full((16,), 1.0, jnp.float32)
    plsc.subcore_barrier()

    w = c * NS + s
    def _hist_pass(hist_s):
        def _start(j, k):
            pltpu.make_async_copy(buf_v.at[pl.ds(0, B)],
                                  hist_s.at[idx_v.at[j]],
                                  dsems[k]).start(add=True)

        def _wait(j, k):
            pltpu.make_async_copy(buf_v.at[pl.ds(0, B)],
                                  hist_s.at[idx_v.at[j]],
                                  dsems[k]).wait()

        for k in range(4):
            _start(k, k)

        def _quad(t, carry):
            j0 = 4 * t + 4
            for k in range(4):
                _wait(j0 + k - 4, k)
                _start(j0 + k, k)
            return carry

        _nq = (RPW - 4) // 4
        lax.fori_loop(0, _nq, _quad, 0)
        for _j in range(4 * _nq + 4, RPW):
            _wait(_j - 4, _j % 4)
            _start(_j, _j % 4)
        for _j in range(RPW - 4, RPW):
            _wait(_j, _j % 4)

    pltpu.sync_copy(src_hbm.at[w], idx_v)
    _hist_pass(hout_s)
    pltpu.sync_copy(dst_hbm.at[w], idx_v)
    _hist_pass(hin_s)
    plsc.subcore_barrier()

    pltpu.sync_copy(hout_s.at[pl.ds(s * CH, CH)], buf_v.at[pl.ds(0, CH)])

    @pl.when(c == 0)
    def _():
        pltpu.sync_copy(buf_v.at[pl.ds(0, CH)], dego0_hbm.at[pl.ds(s * CH, CH)])

    @pl.when(c == 1)
    def _():
        pltpu.sync_copy(buf_v.at[pl.ds(0, CH)], dego1_hbm.at[pl.ds(s * CH, CH)])

    pltpu.sync_copy(hin_s.at[pl.ds(s * CH, CH)], buf_v.at[pl.ds(0, CH)])

    @pl.when(c == 0)
    def _():
        pltpu.sync_copy(buf_v.at[pl.ds(0, CH)], degi0_hbm.at[pl.ds(s * CH, CH)])

    @pl.when(c == 1)
    def _():
        pltpu.sync_copy(buf_v.at[pl.ds(0, CH)], degi1_hbm.at[pl.ds(s * CH, CH)])


@functools.partial(
    pl.kernel,
    out_type=jax.ShapeDtypeStruct((NC, NPAD, D2), jnp.float32),
    mesh=_mesh,
    scratch_types=[
        pltpu.VMEM((RPT * B,), jnp.int32),
        pltpu.VMEM((RPT, B), jnp.int32),
        pltpu.VMEM((B, D2), jnp.float32),
        pltpu.VMEM((B, D2), jnp.float32),
        pltpu.VMEM((B, D2), jnp.float32),
        pltpu.SemaphoreType.DMA,
        pltpu.SemaphoreType.DMA,
        pltpu.SemaphoreType.DMA,
        (pltpu.SemaphoreType.DMA, pltpu.SemaphoreType.DMA,
         pltpu.SemaphoreType.DMA),
        pltpu.VMEM_SHARED((NPAD, D2), jnp.float32),
    ],
    compiler_params=pltpu.CompilerParams(use_tc_tiling_on_sc=False),
)
def _sc_aggregate(h_hbm, srcf_hbm, dst_hbm, out_hbm, sidx, didx,
                  r_a, r_b, r_c, sem_a, sem_b, sem_c, ssems, agg_s):
    c = lax.axis_index("c")
    s = lax.axis_index("s")
    for r in range(B):
        for k in range(D2 // 16):
            r_a[r, pl.ds(k * 16, 16)] = jnp.zeros((16,), jnp.float32)
    for t in range(CH // ZC):
        pltpu.sync_copy(r_a.at[pl.ds(0, ZC)],
                        agg_s.at[pl.ds(s * CH + t * ZC, ZC)])
    pltpu.sync_copy(r_a.at[pl.ds(0, ZREM)],
                    agg_s.at[pl.ds(s * CH + (CH // ZC) * ZC, ZREM)])
    plsc.subcore_barrier()

    pltpu.sync_copy(srcf_hbm.at[s], sidx)
    pltpu.sync_copy(dst_hbm.at[s], didx)

    cbias = jnp.full((16,), c * N, jnp.int32)

    def _adj(i, carry):
        sidx[pl.ds(i * 16, 16)] = sidx[pl.ds(i * 16, 16)] + cbias
        return carry

    lax.fori_loop(0, RPT * B // 16, _adj, 0)

    _bufs = (r_a, r_b, r_c)
    _gsems = (sem_a, sem_b, sem_c)

    def _gather(j, x):
        pltpu.async_copy(h_hbm.at[sidx.at[pl.ds(j * B, B)]], _bufs[x],
                         _gsems[x])

    def _gwait(j, x):
        pltpu.make_async_copy(h_hbm.at[sidx.at[pl.ds(j * B, B)]], _bufs[x],
                              _gsems[x]).wait()

    def _sstart(j, x):
        pltpu.make_async_copy(_bufs[x], agg_s.at[didx.at[j]],
                              ssems[x]).start(add=True)

    def _swait(j, x):
        pltpu.make_async_copy(_bufs[x], agg_s.at[didx.at[j]],
                              ssems[x]).wait()

    _gather(0, 0)
    _gwait(0, 0)
    _sstart(0, 0)
    _gather(1, 1)
    _gwait(1, 1)
    _sstart(1, 1)
    _gather(2, 2)

    def _step(j, x):
        _swait(j - 2, (x + 1) % 3)
        _gather(j + 1, (x + 1) % 3)
        _gwait(j, x)
        _sstart(j, x)

    def _tri(t, carry):
        j0 = 3 * t + 2
        _step(j0, 2)
        _step(j0 + 1, 0)
        _step(j0 + 2, 1)
        return carry

    lax.fori_loop(0, (RPT - 4) // 3, _tri, 0)
    for _j in range(3 * ((RPT - 4) // 3) + 2, RPT):
        _x = _j % 3
        _swait(_j - 2, (_x + 1) % 3)
        if _j + 1 < RPT:
            _gather(_j + 1, (_x + 1) % 3)
        _gwait(_j, _x)
        _sstart(_j, _x)
    _swait(RPT - 2, (RPT - 2) % 3)
    _swait(RPT - 1, (RPT - 1) % 3)

    acc = lax.fori_loop(0, 2048, lambda i, a: a + i, jnp.int32(0))
    sidx[pl.ds(0, 16)] = jnp.full((16,), acc, jnp.int32)
    plsc.subcore_barrier()

    for t in range(CH // ZC):
        pltpu.sync_copy(agg_s.at[pl.ds(s * CH + t * ZC, ZC)],
                        r_a.at[pl.ds(0, ZC)])
        pltpu.sync_copy(r_a.at[pl.ds(0, ZC)],
                        out_hbm.at[c, pl.ds(s * CH + t * ZC, ZC)])
    pltpu.sync_copy(agg_s.at[pl.ds(s * CH + (CH // ZC) * ZC, ZREM)],
                    r_a.at[pl.ds(0, ZREM)])
    pltpu.sync_copy(r_a.at[pl.ds(0, ZREM)],
                    out_hbm.at[c, pl.ds(s * CH + (CH // ZC) * ZC, ZREM)])


_BR = 1000


def _norm(two_col):
    deg = two_col[:, 0:1] + two_col[:, 1:2]
    return jnp.where(deg > 0, lax.rsqrt(deg), 0.0)


def _split_store(o_ref, y):
    o_ref[0, :, :] = y[:, :D2]
    o_ref[1, :, :] = y[:, D2:]


def _mm1_body(f_ref, w_ref, dego_ref, o_ref):
    ns = _norm(dego_ref[...])
    y = jnp.dot(f_ref[...], w_ref[...],
                preferred_element_type=jnp.float32) * ns
    _split_store(o_ref, y)


def _tc_mm1(feat, W1, dego_t):
    return pl.pallas_call(
        _mm1_body,
        grid=(N // _BR,),
        in_specs=[
            pl.BlockSpec((_BR, D), lambda i: (i, 0)),
            pl.BlockSpec((D, D), lambda i: (0, 0)),
            pl.BlockSpec((_BR, NC), lambda i: (i, 0)),
        ],
        out_specs=pl.BlockSpec((NC, _BR, D2), lambda i: (0, i, 0)),
        out_shape=jax.ShapeDtypeStruct((NC, N, D2), jnp.float32),
    )(feat, W1, dego_t)


def _mm2_body(p_ref, degi_ref, dego_ref, b_ref, w_ref, o_ref):
    agg = jnp.concatenate([p_ref[0], p_ref[1]], axis=-1)
    nd = _norm(degi_ref[...])
    x = jnp.maximum(agg * nd + b_ref[...], 0.0)
    ns = _norm(dego_ref[...])
    y = jnp.dot(x, w_ref[...], preferred_element_type=jnp.float32) * ns
    _split_store(o_ref, y)


def _tc_mm2(P, degi_t, dego_t, b1, W2):
    return pl.pallas_call(
        _mm2_body,
        grid=(N // _BR,),
        in_specs=[
            pl.BlockSpec((NC, _BR, D2), lambda i: (0, i, 0)),
            pl.BlockSpec((_BR, NC), lambda i: (i, 0)),
            pl.BlockSpec((_BR, NC), lambda i: (i, 0)),
            pl.BlockSpec((1, D), lambda i: (0, 0)),
            pl.BlockSpec((D, D), lambda i: (0, 0)),
        ],
        out_specs=pl.BlockSpec((NC, _BR, D2), lambda i: (0, i, 0)),
        out_shape=jax.ShapeDtypeStruct((NC, N, D2), jnp.float32),
    )(P, degi_t, dego_t, b1, W2)


def _ep_body(p_ref, degi_ref, b_ref, o_ref):
    agg = jnp.concatenate([p_ref[0], p_ref[1]], axis=-1)
    nd = _norm(degi_ref[...])
    o_ref[...] = jnp.maximum(agg * nd + b_ref[...], 0.0)


def _tc_ep(P, degi_t, b2):
    return pl.pallas_call(
        _ep_body,
        grid=(N // _BR,),
        in_specs=[
            pl.BlockSpec((NC, _BR, D2), lambda i: (0, i, 0)),
            pl.BlockSpec((_BR, NC), lambda i: (i, 0)),
            pl.BlockSpec((1, D), lambda i: (0, 0)),
        ],
        out_specs=pl.BlockSpec((_BR, D), lambda i: (i, 0)),
        out_shape=jax.ShapeDtypeStruct((N, D), jnp.float32),
    )(P, degi_t, b2)


def kernel(feat, edge_index, efeat, W1, b1, W2, b2):
    del efeat
    srcf = edge_index[0].reshape(NS, RPT * B)
    dst3d = edge_index[1].reshape(NS, RPT, B)
    src_deg = edge_index[0].reshape(NW, RPW, B)
    dst_deg = edge_index[1].reshape(NW, RPW, B)

    dego0, dego1, degi0, degi1 = _sc_degrees(src_deg, dst_deg)
    dego_t = jnp.stack([dego0, dego1], axis=1)
    degi_t = jnp.stack([degi0, degi1], axis=1)

    h1 = _tc_mm1(feat, W1, dego_t)
    P1 = _sc_aggregate(h1.reshape(NC * N, D2), srcf, dst3d)
    h2 = _tc_mm2(P1, degi_t, dego_t, b1.reshape(1, D), W2)
    P2 = _sc_aggregate(h2.reshape(NC * N, D2), srcf, dst3d)
    return _tc_ep(P2, degi_t, b2.reshape(1, D))


# speedup vs baseline: 10.6083x; 1.0259x over previous
import functools

import jax
import jax.numpy as jnp
from jax import lax
from jax.experimental import pallas as pl
from jax.experimental.pallas import tpu as pltpu
from jax.experimental.pallas import tpu_sc as plsc

N = 10000
E = 320000
D = 128
D2 = D // 2

NC = 2
NS = 16
NW = NC * NS
B = 80
NROWS = E // B
RPT = NROWS // NS
RPW = NROWS // NW
NPAD = 10112
CH = NPAD // NS
ZC = 40
ZREM = CH - (CH // ZC) * ZC
T_TRI = (RPT - 2) // 3

_mesh = plsc.VectorSubcoreMesh(core_axis_name="c", subcore_axis_name="s")


@functools.partial(
    pl.kernel,
    out_type=(
        jax.ShapeDtypeStruct((NPAD,), jnp.float32),
        jax.ShapeDtypeStruct((NPAD,), jnp.float32),
        jax.ShapeDtypeStruct((NPAD,), jnp.float32),
        jax.ShapeDtypeStruct((NPAD,), jnp.float32),
    ),
    mesh=_mesh,
    scratch_types=[
        pltpu.VMEM((RPW, B), jnp.int32),
        pltpu.VMEM((640,), jnp.float32),
        (pltpu.SemaphoreType.DMA, pltpu.SemaphoreType.DMA,
         pltpu.SemaphoreType.DMA, pltpu.SemaphoreType.DMA),
        pltpu.VMEM_SHARED((NPAD,), jnp.float32),
        pltpu.VMEM_SHARED((NPAD,), jnp.float32),
    ],
    compiler_params=pltpu.CompilerParams(use_tc_tiling_on_sc=False),
)
def _sc_degrees(src_hbm, dst_hbm, dego0_hbm, dego1_hbm, degi0_hbm, degi1_hbm,
                idx_v, buf_v, dsems, hout_s, hin_s):
    c = lax.axis_index("c")
    s = lax.axis_index("s")
    for k in range(40):
        buf_v[pl.ds(k * 16, 16)] = jnp.zeros((16,), jnp.float32)
    pltpu.sync_copy(buf_v.at[pl.ds(0, CH)], hout_s.at[pl.ds(s * CH, CH)])
    pltpu.sync_copy(buf_v.at[pl.ds(0, CH)], hin_s.at[pl.ds(s * CH, CH)])
    for k in range((B + 15) // 16):
        buf_v[pl.ds(k * 16, 16)] = jnp.full((16,), 1.0, jnp.float32)
    plsc.subcore_barrier()

    w = c * NS + s
    def _hist_pass(hist_s):
        def _start(j, k):
            pltpu.make_async_copy(buf_v.at[pl.ds(0, B)],
                                  hist_s.at[idx_v.at[j]],
                                  dsems[k]).start(add=True)

        def _wait(j, k):
            pltpu.make_async_copy(buf_v.at[pl.ds(0, B)],
                                  hist_s.at[idx_v.at[j]],
                                  dsems[k]).wait()

        for k in range(4):
            _start(k, k)

        def _quad(t, carry):
            j0 = 4 * t + 4
            for k in range(4):
                _wait(j0 + k - 4, k)
                _start(j0 + k, k)
            return carry

        _nq = (RPW - 4) // 4
        lax.fori_loop(0, _nq, _quad, 0)
        for _j in range(4 * _nq + 4, RPW):
            _wait(_j - 4, _j % 4)
            _start(_j, _j % 4)
        for _j in range(RPW - 4, RPW):
            _wait(_j, _j % 4)

    pltpu.sync_copy(src_hbm.at[w // 2, pl.ds((w % 2) * RPW, RPW)], idx_v)
    _hist_pass(hout_s)
    pltpu.sync_copy(dst_hbm.at[w // 2, pl.ds((w % 2) * RPW, RPW)], idx_v)
    _hist_pass(hin_s)
    plsc.subcore_barrier()

    pltpu.sync_copy(hout_s.at[pl.ds(s * CH, CH)], buf_v.at[pl.ds(0, CH)])

    @pl.when(c == 0)
    def _():
        pltpu.sync_copy(buf_v.at[pl.ds(0, CH)], dego0_hbm.at[pl.ds(s * CH, CH)])

    @pl.when(c == 1)
    def _():
        pltpu.sync_copy(buf_v.at[pl.ds(0, CH)], dego1_hbm.at[pl.ds(s * CH, CH)])

    pltpu.sync_copy(hin_s.at[pl.ds(s * CH, CH)], buf_v.at[pl.ds(0, CH)])

    @pl.when(c == 0)
    def _():
        pltpu.sync_copy(buf_v.at[pl.ds(0, CH)], degi0_hbm.at[pl.ds(s * CH, CH)])

    @pl.when(c == 1)
    def _():
        pltpu.sync_copy(buf_v.at[pl.ds(0, CH)], degi1_hbm.at[pl.ds(s * CH, CH)])


@functools.partial(
    pl.kernel,
    out_type=jax.ShapeDtypeStruct((NC, NPAD, D2), jnp.float32),
    mesh=_mesh,
    scratch_types=[
        pltpu.VMEM((RPT * B,), jnp.int32),
        pltpu.VMEM((RPT, B), jnp.int32),
        pltpu.VMEM((B, D2), jnp.float32),
        pltpu.VMEM((B, D2), jnp.float32),
        pltpu.VMEM((B, D2), jnp.float32),
        pltpu.SemaphoreType.DMA,
        pltpu.SemaphoreType.DMA,
        pltpu.SemaphoreType.DMA,
        (pltpu.SemaphoreType.DMA, pltpu.SemaphoreType.DMA,
         pltpu.SemaphoreType.DMA),
        pltpu.VMEM_SHARED((NPAD, D2), jnp.float32),
    ],
    compiler_params=pltpu.CompilerParams(use_tc_tiling_on_sc=False),
)
def _sc_aggregate(h_hbm, srcf_hbm, dst_hbm, out_hbm, sidx, didx,
                  r_a, r_b, r_c, sem_a, sem_b, sem_c, ssems, agg_s):
    c = lax.axis_index("c")
    s = lax.axis_index("s")
    for r in range(B):
        for k in range(D2 // 16):
            r_a[r, pl.ds(k * 16, 16)] = jnp.zeros((16,), jnp.float32)
    for t in range(CH // ZC):
        pltpu.sync_copy(r_a.at[pl.ds(0, ZC)],
                        agg_s.at[pl.ds(s * CH + t * ZC, ZC)])
    pltpu.sync_copy(r_a.at[pl.ds(0, ZREM)],
                    agg_s.at[pl.ds(s * CH + (CH // ZC) * ZC, ZREM)])
    plsc.subcore_barrier()

    pltpu.sync_copy(srcf_hbm.at[s], sidx)
    pltpu.sync_copy(dst_hbm.at[s], didx)

    cbias = jnp.full((16,), c * N, jnp.int32)

    def _adj(i, carry):
        sidx[pl.ds(i * 16, 16)] = sidx[pl.ds(i * 16, 16)] + cbias
        return carry

    lax.fori_loop(0, RPT * B // 16, _adj, 0)

    _bufs = (r_a, r_b, r_c)
    _gsems = (sem_a, sem_b, sem_c)

    def _gather(j, x):
        pltpu.async_copy(h_hbm.at[sidx.at[pl.ds(j * B, B)]], _bufs[x],
                         _gsems[x])

    def _gwait(j, x):
        pltpu.make_async_copy(h_hbm.at[sidx.at[pl.ds(j * B, B)]], _bufs[x],
                              _gsems[x]).wait()

    def _sstart(j, x):
        pltpu.make_async_copy(_bufs[x], agg_s.at[didx.at[j]],
                              ssems[x]).start(add=True)

    def _swait(j, x):
        pltpu.make_async_copy(_bufs[x], agg_s.at[didx.at[j]],
                              ssems[x]).wait()

    _gather(0, 0)
    _gwait(0, 0)
    _sstart(0, 0)
    _gather(1, 1)
    _gwait(1, 1)
    _sstart(1, 1)
    _gather(2, 2)

    def _step(j, x):
        _swait(j - 2, (x + 1) % 3)
        _gather(j + 1, (x + 1) % 3)
        _gwait(j, x)
        _sstart(j, x)

    def _tri(t, carry):
        j0 = 3 * t + 2
        _step(j0, 2)
        _step(j0 + 1, 0)
        _step(j0 + 2, 1)
        return carry

    lax.fori_loop(0, (RPT - 4) // 3, _tri, 0)
    for _j in range(3 * ((RPT - 4) // 3) + 2, RPT):
        _x = _j % 3
        _swait(_j - 2, (_x + 1) % 3)
        if _j + 1 < RPT:
            _gather(_j + 1, (_x + 1) % 3)
        _gwait(_j, _x)
        _sstart(_j, _x)
    _swait(RPT - 2, (RPT - 2) % 3)
    _swait(RPT - 1, (RPT - 1) % 3)

    plsc.subcore_barrier()

    for t in range(CH // ZC):
        pltpu.sync_copy(agg_s.at[pl.ds(s * CH + t * ZC, ZC)],
                        r_a.at[pl.ds(0, ZC)])
        pltpu.sync_copy(r_a.at[pl.ds(0, ZC)],
                        out_hbm.at[c, pl.ds(s * CH + t * ZC, ZC)])
    pltpu.sync_copy(agg_s.at[pl.ds(s * CH + (CH // ZC) * ZC, ZREM)],
                    r_a.at[pl.ds(0, ZREM)])
    pltpu.sync_copy(r_a.at[pl.ds(0, ZREM)],
                    out_hbm.at[c, pl.ds(s * CH + (CH // ZC) * ZC, ZREM)])


_BR = 2000


def _norm(two_col):
    deg = two_col[:, 0:1] + two_col[:, 1:2]
    return jnp.where(deg > 0, lax.rsqrt(deg), 0.0)


def _split_store(o_ref, y):
    o_ref[0, :, :] = y[:, :D2]
    o_ref[1, :, :] = y[:, D2:]


def _mm1_body(f_ref, w_ref, dego_ref, o_ref):
    ns = _norm(dego_ref[...])
    y = jnp.dot(f_ref[...], w_ref[...],
                preferred_element_type=jnp.float32) * ns
    _split_store(o_ref, y)


def _tc_mm1(feat, W1, dego_t):
    return pl.pallas_call(
        _mm1_body,
        grid=(N // _BR,),
        in_specs=[
            pl.BlockSpec((_BR, D), lambda i: (i, 0)),
            pl.BlockSpec((D, D), lambda i: (0, 0)),
            pl.BlockSpec((_BR, NC), lambda i: (i, 0)),
        ],
        out_specs=pl.BlockSpec((NC, _BR, D2), lambda i: (0, i, 0)),
        out_shape=jax.ShapeDtypeStruct((NC, N, D2), jnp.float32),
    )(feat, W1, dego_t)


def _mm2_body(p_ref, degi_ref, dego_ref, b_ref, w_ref, o_ref):
    agg = jnp.concatenate([p_ref[0], p_ref[1]], axis=-1)
    nd = _norm(degi_ref[...])
    x = jnp.maximum(agg * nd + b_ref[...], 0.0)
    ns = _norm(dego_ref[...])
    y = jnp.dot(x, w_ref[...], preferred_element_type=jnp.float32) * ns
    _split_store(o_ref, y)


def _tc_mm2(P, degi_t, dego_t, b1, W2):
    return pl.pallas_call(
        _mm2_body,
        grid=(N // _BR,),
        in_specs=[
            pl.BlockSpec((NC, _BR, D2), lambda i: (0, i, 0)),
            pl.BlockSpec((_BR, NC), lambda i: (i, 0)),
            pl.BlockSpec((_BR, NC), lambda i: (i, 0)),
            pl.BlockSpec((1, D), lambda i: (0, 0)),
            pl.BlockSpec((D, D), lambda i: (0, 0)),
        ],
        out_specs=pl.BlockSpec((NC, _BR, D2), lambda i: (0, i, 0)),
        out_shape=jax.ShapeDtypeStruct((NC, N, D2), jnp.float32),
    )(P, degi_t, dego_t, b1, W2)


def _ep_body(p_ref, degi_ref, b_ref, o_ref):
    agg = jnp.concatenate([p_ref[0], p_ref[1]], axis=-1)
    nd = _norm(degi_ref[...])
    o_ref[...] = jnp.maximum(agg * nd + b_ref[...], 0.0)


def _tc_ep(P, degi_t, b2):
    return pl.pallas_call(
        _ep_body,
        grid=(N // _BR,),
        in_specs=[
            pl.BlockSpec((NC, _BR, D2), lambda i: (0, i, 0)),
            pl.BlockSpec((_BR, NC), lambda i: (i, 0)),
            pl.BlockSpec((1, D), lambda i: (0, 0)),
        ],
        out_specs=pl.BlockSpec((_BR, D), lambda i: (i, 0)),
        out_shape=jax.ShapeDtypeStruct((N, D), jnp.float32),
    )(P, degi_t, b2)


def kernel(feat, edge_index, efeat, W1, b1, W2, b2):
    del efeat
    srcf = edge_index[0].reshape(NS, RPT * B)
    src3d = edge_index[0].reshape(NS, RPT, B)
    dst3d = edge_index[1].reshape(NS, RPT, B)

    dego0, dego1, degi0, degi1 = _sc_degrees(src3d, dst3d)
    dego_t = jnp.stack([dego0, dego1], axis=1)
    degi_t = jnp.stack([degi0, degi1], axis=1)

    h1 = _tc_mm1(feat, W1, dego_t)
    P1 = _sc_aggregate(h1.reshape(NC * N, D2), srcf, dst3d)
    h2 = _tc_mm2(P1, degi_t, dego_t, b1.reshape(1, D), W2)
    P2 = _sc_aggregate(h2.reshape(NC * N, D2), srcf, dst3d)
    return _tc_ep(P2, degi_t, b2.reshape(1, D))


# speedup vs baseline: 11.2565x; 1.0611x over previous
import functools

import jax
import jax.numpy as jnp
from jax import lax
from jax.experimental import pallas as pl
from jax.experimental.pallas import tpu as pltpu
from jax.experimental.pallas import tpu_sc as plsc

N = 10000
E = 320000
D = 128
D2 = D // 2

NC = 2
NS = 16
NW = NC * NS
B = 80
NROWS = E // B
RPT = NROWS // NS
RPW = NROWS // NW
NPAD = 10112
CH = NPAD // NS
ZC = 40
ZREM = CH - (CH // ZC) * ZC
T_TRI = (RPT - 2) // 3

_mesh = plsc.VectorSubcoreMesh(core_axis_name="c", subcore_axis_name="s")


@functools.partial(
    pl.kernel,
    out_type=(
        jax.ShapeDtypeStruct((NPAD,), jnp.float32),
        jax.ShapeDtypeStruct((NPAD,), jnp.float32),
        jax.ShapeDtypeStruct((NPAD,), jnp.float32),
        jax.ShapeDtypeStruct((NPAD,), jnp.float32),
    ),
    mesh=_mesh,
    scratch_types=[
        pltpu.VMEM((RPW, B), jnp.int32),
        pltpu.VMEM((640,), jnp.float32),
        (pltpu.SemaphoreType.DMA, pltpu.SemaphoreType.DMA,
         pltpu.SemaphoreType.DMA, pltpu.SemaphoreType.DMA),
        pltpu.VMEM_SHARED((NPAD,), jnp.float32),
        pltpu.VMEM_SHARED((NPAD,), jnp.float32),
    ],
    compiler_params=pltpu.CompilerParams(use_tc_tiling_on_sc=False),
)
def _sc_degrees(src_hbm, dst_hbm, dego0_hbm, dego1_hbm, degi0_hbm, degi1_hbm,
                idx_v, buf_v, dsems, hout_s, hin_s):
    c = lax.axis_index("c")
    s = lax.axis_index("s")
    for k in range(40):
        buf_v[pl.ds(k * 16, 16)] = jnp.zeros((16,), jnp.float32)
    pltpu.sync_copy(buf_v.at[pl.ds(0, CH)], hout_s.at[pl.ds(s * CH, CH)])
    pltpu.sync_copy(buf_v.at[pl.ds(0, CH)], hin_s.at[pl.ds(s * CH, CH)])
    for k in range((B + 15) // 16):
        buf_v[pl.ds(k * 16, 16)] = jnp.full((16,), 1.0, jnp.float32)
    plsc.subcore_barrier()

    w = c * NS + s
    def _hist_pass(hist_s):
        def _start(j, k):
            pltpu.make_async_copy(buf_v.at[pl.ds(0, B)],
                                  hist_s.at[idx_v.at[j]],
                                  dsems[k]).start(add=True)

        def _wait(j, k):
            pltpu.make_async_copy(buf_v.at[pl.ds(0, B)],
                                  hist_s.at[idx_v.at[j]],
                                  dsems[k]).wait()

        for k in range(4):
            _start(k, k)

        def _quad(t, carry):
            j0 = 4 * t + 4
            for k in range(4):
                _wait(j0 + k - 4, k)
                _start(j0 + k, k)
            return carry

        _nq = (RPW - 4) // 4
        lax.fori_loop(0, _nq, _quad, 0)
        for _j in range(4 * _nq + 4, RPW):
            _wait(_j - 4, _j % 4)
            _start(_j, _j % 4)
        for _j in range(RPW - 4, RPW):
            _wait(_j, _j % 4)

    pltpu.sync_copy(src_hbm.at[w // 2, pl.ds((w % 2) * RPW, RPW)], idx_v)
    _hist_pass(hout_s)
    pltpu.sync_copy(dst_hbm.at[w // 2, pl.ds((w % 2) * RPW, RPW)], idx_v)
    _hist_pass(hin_s)
    plsc.subcore_barrier()

    pltpu.sync_copy(hout_s.at[pl.ds(s * CH, CH)], buf_v.at[pl.ds(0, CH)])

    @pl.when(c == 0)
    def _():
        pltpu.sync_copy(buf_v.at[pl.ds(0, CH)], dego0_hbm.at[pl.ds(s * CH, CH)])

    @pl.when(c == 1)
    def _():
        pltpu.sync_copy(buf_v.at[pl.ds(0, CH)], dego1_hbm.at[pl.ds(s * CH, CH)])

    pltpu.sync_copy(hin_s.at[pl.ds(s * CH, CH)], buf_v.at[pl.ds(0, CH)])

    @pl.when(c == 0)
    def _():
        pltpu.sync_copy(buf_v.at[pl.ds(0, CH)], degi0_hbm.at[pl.ds(s * CH, CH)])

    @pl.when(c == 1)
    def _():
        pltpu.sync_copy(buf_v.at[pl.ds(0, CH)], degi1_hbm.at[pl.ds(s * CH, CH)])


@functools.partial(
    pl.kernel,
    out_type=jax.ShapeDtypeStruct((NC, NPAD, D2), jnp.float32),
    mesh=_mesh,
    scratch_types=[
        pltpu.VMEM((RPT * B,), jnp.int32),
        pltpu.VMEM((RPT, B), jnp.int32),
        pltpu.VMEM((B, D2), jnp.float32),
        pltpu.VMEM((B, D2), jnp.float32),
        pltpu.VMEM((B, D2), jnp.float32),
        pltpu.SemaphoreType.DMA,
        pltpu.SemaphoreType.DMA,
        pltpu.SemaphoreType.DMA,
        (pltpu.SemaphoreType.DMA, pltpu.SemaphoreType.DMA,
         pltpu.SemaphoreType.DMA),
        pltpu.VMEM_SHARED((NPAD, D2), jnp.float32),
    ],
    compiler_params=pltpu.CompilerParams(use_tc_tiling_on_sc=False),
)
def _sc_aggregate(h_hbm, srcf_hbm, dst_hbm, out_hbm, sidx, didx,
                  r_a, r_b, r_c, sem_a, sem_b, sem_c, ssems, agg_s):
    c = lax.axis_index("c")
    s = lax.axis_index("s")
    for r in range(B):
        for k in range(D2 // 16):
            r_a[r, pl.ds(k * 16, 16)] = jnp.zeros((16,), jnp.float32)
    for t in range(CH // ZC):
        pltpu.sync_copy(r_a.at[pl.ds(0, ZC)],
                        agg_s.at[pl.ds(s * CH + t * ZC, ZC)])
    pltpu.sync_copy(r_a.at[pl.ds(0, ZREM)],
                    agg_s.at[pl.ds(s * CH + (CH // ZC) * ZC, ZREM)])
    plsc.subcore_barrier()

    pltpu.sync_copy(srcf_hbm.at[s], sidx)
    pltpu.sync_copy(dst_hbm.at[s], didx)

    cbias = jnp.full((16,), c, jnp.int32)

    def _adj(i, carry):
        v = sidx[pl.ds(i * 16, 16)]
        sidx[pl.ds(i * 16, 16)] = v + v + cbias
        return carry

    lax.fori_loop(0, RPT * B // 16, _adj, 0)

    _bufs = (r_a, r_b, r_c)
    _gsems = (sem_a, sem_b, sem_c)

    def _gather(j, x):
        pltpu.async_copy(h_hbm.at[sidx.at[pl.ds(j * B, B)]], _bufs[x],
                         _gsems[x])

    def _gwait(j, x):
        pltpu.make_async_copy(h_hbm.at[sidx.at[pl.ds(j * B, B)]], _bufs[x],
                              _gsems[x]).wait()

    def _sstart(j, x):
        pltpu.make_async_copy(_bufs[x], agg_s.at[didx.at[j]],
                              ssems[x]).start(add=True)

    def _swait(j, x):
        pltpu.make_async_copy(_bufs[x], agg_s.at[didx.at[j]],
                              ssems[x]).wait()

    _gather(0, 0)
    _gwait(0, 0)
    _sstart(0, 0)
    _gather(1, 1)
    _gwait(1, 1)
    _sstart(1, 1)
    _gather(2, 2)

    def _step(j, x):
        _swait(j - 2, (x + 1) % 3)
        _gather(j + 1, (x + 1) % 3)
        _gwait(j, x)
        _sstart(j, x)

    def _tri(t, carry):
        j0 = 3 * t + 2
        _step(j0, 2)
        _step(j0 + 1, 0)
        _step(j0 + 2, 1)
        return carry

    lax.fori_loop(0, (RPT - 4) // 3, _tri, 0)
    for _j in range(3 * ((RPT - 4) // 3) + 2, RPT):
        _x = _j % 3
        _swait(_j - 2, (_x + 1) % 3)
        if _j + 1 < RPT:
            _gather(_j + 1, (_x + 1) % 3)
        _gwait(_j, _x)
        _sstart(_j, _x)
    _swait(RPT - 2, (RPT - 2) % 3)
    _swait(RPT - 1, (RPT - 1) % 3)

    plsc.subcore_barrier()

    for t in range(CH // ZC):
        pltpu.sync_copy(agg_s.at[pl.ds(s * CH + t * ZC, ZC)],
                        r_a.at[pl.ds(0, ZC)])
        pltpu.sync_copy(r_a.at[pl.ds(0, ZC)],
                        out_hbm.at[c, pl.ds(s * CH + t * ZC, ZC)])
    pltpu.sync_copy(agg_s.at[pl.ds(s * CH + (CH // ZC) * ZC, ZREM)],
                    r_a.at[pl.ds(0, ZREM)])
    pltpu.sync_copy(r_a.at[pl.ds(0, ZREM)],
                    out_hbm.at[c, pl.ds(s * CH + (CH // ZC) * ZC, ZREM)])


_BR = 2000


def _norm(two_col):
    deg = two_col[:, 0:1] + two_col[:, 1:2]
    return jnp.where(deg > 0, lax.rsqrt(deg), 0.0)


def _mm1_body(f_ref, w_ref, dego_ref, o_ref):
    ns = _norm(dego_ref[...])
    o_ref[...] = jnp.dot(f_ref[...], w_ref[...],
                         preferred_element_type=jnp.float32) * ns


def _tc_mm1(feat, W1, dego_t):
    return pl.pallas_call(
        _mm1_body,
        grid=(N // _BR,),
        in_specs=[
            pl.BlockSpec((_BR, D), lambda i: (i, 0)),
            pl.BlockSpec((D, D), lambda i: (0, 0)),
            pl.BlockSpec((_BR, NC), lambda i: (i, 0)),
        ],
        out_specs=pl.BlockSpec((_BR, D), lambda i: (i, 0)),
        out_shape=jax.ShapeDtypeStruct((N, D), jnp.float32),
    )(feat, W1, dego_t)


def _mm2_body(p_ref, degi_ref, dego_ref, b_ref, w_ref, o_ref):
    agg = jnp.concatenate([p_ref[0], p_ref[1]], axis=-1)
    nd = _norm(degi_ref[...])
    x = jnp.maximum(agg * nd + b_ref[...], 0.0)
    ns = _norm(dego_ref[...])
    o_ref[...] = jnp.dot(x, w_ref[...], preferred_element_type=jnp.float32) * ns


def _tc_mm2(P, degi_t, dego_t, b1, W2):
    return pl.pallas_call(
        _mm2_body,
        grid=(N // _BR,),
        in_specs=[
            pl.BlockSpec((NC, _BR, D2), lambda i: (0, i, 0)),
            pl.BlockSpec((_BR, NC), lambda i: (i, 0)),
            pl.BlockSpec((_BR, NC), lambda i: (i, 0)),
            pl.BlockSpec((1, D), lambda i: (0, 0)),
            pl.BlockSpec((D, D), lambda i: (0, 0)),
        ],
        out_specs=pl.BlockSpec((_BR, D), lambda i: (i, 0)),
        out_shape=jax.ShapeDtypeStruct((N, D), jnp.float32),
    )(P, degi_t, dego_t, b1, W2)


def _ep_body(p_ref, degi_ref, b_ref, o_ref):
    agg = jnp.concatenate([p_ref[0], p_ref[1]], axis=-1)
    nd = _norm(degi_ref[...])
    o_ref[...] = jnp.maximum(agg * nd + b_ref[...], 0.0)


def _tc_ep(P, degi_t, b2):
    return pl.pallas_call(
        _ep_body,
        grid=(N // _BR,),
        in_specs=[
            pl.BlockSpec((NC, _BR, D2), lambda i: (0, i, 0)),
            pl.BlockSpec((_BR, NC), lambda i: (i, 0)),
            pl.BlockSpec((1, D), lambda i: (0, 0)),
        ],
        out_specs=pl.BlockSpec((_BR, D), lambda i: (i, 0)),
        out_shape=jax.ShapeDtypeStruct((N, D), jnp.float32),
    )(P, degi_t, b2)


def kernel(feat, edge_index, efeat, W1, b1, W2, b2):
    del efeat
    srcf = edge_index[0].reshape(NS, RPT * B)
    src3d = edge_index[0].reshape(NS, RPT, B)
    dst3d = edge_index[1].reshape(NS, RPT, B)

    dego0, dego1, degi0, degi1 = _sc_degrees(src3d, dst3d)
    dego_t = jnp.stack([dego0, dego1], axis=1)
    degi_t = jnp.stack([degi0, degi1], axis=1)

    h1 = _tc_mm1(feat, W1, dego_t)
    P1 = _sc_aggregate(h1.reshape(NC * N, D2), srcf, dst3d)
    h2 = _tc_mm2(P1, degi_t, dego_t, b1.reshape(1, D), W2)
    P2 = _sc_aggregate(h2.reshape(NC * N, D2), srcf, dst3d)
    return _tc_ep(P2, degi_t, b2.reshape(1, D))
